# Initial kernel scaffold; baseline (speedup 1.0000x reference)
import jax
import jax.numpy as jnp
from jax.experimental import pallas as pl


def kernel(x, edge_index, edge_attr, W1, b1, W2, b2, W_mu, b_mu, W_std, b_std):
    raise NotImplementedError("write your pallas kernel here")


# speedup vs baseline: 5.3957x; 5.3957x over previous
import functools

import jax
import jax.numpy as jnp
from jax import lax
from jax.experimental import pallas as pl
from jax.experimental.pallas import tpu as pltpu
from jax.experimental.pallas import tpu_sc as plsc

_N = 10000
_NP = 10240
_E = 320000
_D_IN = 128
_H1 = 256
_H2 = 128

_NT = 16
_NC = 2
_T = 20992
_E2P = _NT * _T
_CH = 256
_NCHUNK = _T // _CH
_RPT = _T // 128
_NODES_PER_TILE = _NP // _NT

_f32 = jnp.float32
_i32 = jnp.int32


def _bcast16(s):
  return jnp.broadcast_to(s, (16,))


def _rsqrt16(d):
  x = jnp.maximum(d, 1e-12)
  i = lax.bitcast_convert_type(x, _i32)
  i = jnp.int32(0x5F3759DF) - (i >> 1)
  y = lax.bitcast_convert_type(i, _f32)
  for _ in range(3):
    y = y * (1.5 - 0.5 * x * y * y)
  return jnp.where(d > 0, y, 0.0)


def _norm_body(src_h, dst_h, ew_h, norm_h,
               deg_sp, dinv_sp, sb, db, wb, nb, dinvb, slab, sem):
  cid = lax.axis_index("c")
  wid = lax.axis_index("s")

  zero16 = jnp.zeros((16,), _f32)
  def zbody(i, _):
    slab[pl.ds(i * 16, 16)] = zero16
    return 0
  lax.fori_loop(0, _NODES_PER_TILE // 16, zbody, 0)
  pltpu.sync_copy(slab, deg_sp.at[pl.ds(wid * _NODES_PER_TILE,
                                        _NODES_PER_TILE)])
  plsc.subcore_barrier()

  def deg_body(c, _):
    row = wid * _RPT + 4 * c
    pltpu.sync_copy(dst_h.at[pl.ds(row, 4)], db)
    pltpu.sync_copy(ew_h.at[pl.ds(row, 4)], wb)
    for r in range(4):
      pltpu.sync_copy(wb.at[r], deg_sp.at[db.at[r]], add=True)
    return 0
  lax.fori_loop(0, _RPT // 4, deg_body, 0)
  plsc.subcore_barrier()

  pltpu.sync_copy(deg_sp.at[pl.ds(wid * _NODES_PER_TILE, _NODES_PER_TILE)],
                  slab)
  def rbody(i, _):
    d = slab[pl.ds(i * 16, 16)]
    slab[pl.ds(i * 16, 16)] = _rsqrt16(d)
    return 0
  lax.fori_loop(0, _NODES_PER_TILE // 16, rbody, 0)
  pltpu.sync_copy(slab, dinv_sp.at[pl.ds(wid * _NODES_PER_TILE,
                                         _NODES_PER_TILE)])
  plsc.subcore_barrier()
  pltpu.sync_copy(dinv_sp, dinvb)

  w32 = wid * _NC + cid
  rows_per_w = (_E2P // 128) // (_NT * _NC)
  def norm_chunk(c, _):
    row = w32 * rows_per_w + 2 * c
    pltpu.sync_copy(src_h.at[pl.ds(row, 2)], sb)
    pltpu.sync_copy(dst_h.at[pl.ds(row, 2)], db.at[pl.ds(0, 2)])
    pltpu.sync_copy(ew_h.at[pl.ds(row, 2)], wb.at[pl.ds(0, 2)])
    for r in range(2):
      for k in range(8):
        sl = pl.ds(k * 16, 16)
        s = sb[r, sl]
        d = db[r, sl]
        w = wb[r, sl]
        gs = plsc.load_gather(dinvb, [s])
        gd = plsc.load_gather(dinvb, [d])
        nb[r, sl] = gs * w * gd
    pltpu.sync_copy(nb, norm_h.at[pl.ds(row, 2)])
    return 0
  lax.fori_loop(0, rows_per_w // 2, norm_chunk, 0)


def _make_norm_kernel():
  mesh = plsc.VectorSubcoreMesh(core_axis_name="c", subcore_axis_name="s")
  nrows = _E2P // 128
  return pl.kernel(
      _norm_body,
      out_type=jax.ShapeDtypeStruct((nrows, 128), _f32),
      mesh=mesh,
      compiler_params=pltpu.CompilerParams(needs_layout_passes=False),
      scratch_types=[
          pltpu.VMEM_SHARED((_NP,), _f32),
          pltpu.VMEM_SHARED((_NP,), _f32),
          pltpu.VMEM((2, 128), _i32),
          pltpu.VMEM((4, 128), _i32),
          pltpu.VMEM((4, 128), _f32),
          pltpu.VMEM((2, 128), _f32),
          pltpu.VMEM((_NP,), _f32),
          pltpu.VMEM((_NODES_PER_TILE,), _f32),
          pltpu.SemaphoreType.DMA,
      ],
  )


def _conv_body(relu, h2_h, src_h, dst_h, norm_h, bias_h, ol_h, or_h,
               acc, rowsb, sb, db, ib, nflat, bb, sem, sem2):
  cid = lax.axis_index("c")
  wid = lax.axis_index("s")
  cid16 = _bcast16(cid)

  zero16 = jnp.zeros((16,), _f32)
  def zbody(i, _):
    for j in range(8):
      rowsb[i, pl.ds(j * 16, 16)] = zero16
    return 0
  lax.fori_loop(0, 128, zbody, 0)
  nbase = wid * _NODES_PER_TILE
  for k in range(5):
    pltpu.sync_copy(rowsb.at[pl.ds(0, 128)],
                    acc.at[pl.ds(nbase + 128 * k, 128)])
  plsc.subcore_barrier()

  def chunk_body(c, _):
    row = wid * _RPT + 2 * c
    ebase = wid * _T + _CH * c
    pltpu.sync_copy(src_h.at[pl.ds(row, 2)], sb)
    pltpu.sync_copy(dst_h.at[pl.ds(row, 2)], db)
    pltpu.sync_copy(norm_h.at[pl.ds(ebase, _CH)], nflat)
    for r in range(2):
      for k in range(8):
        sl = pl.ds(k * 16, 16)
        ib[r, sl] = sb[r, sl] * 2 + cid16
    cps = [pltpu.async_copy(h2_h.at[ib.at[r]],
                            rowsb.at[pl.ds(128 * r, 128)], sem)
           for r in range(2)]
    for cp in cps:
      cp.wait()
    def scale_body(e, _):
      nv = plsc.load_gather(nflat, [_bcast16(e)])
      for j in range(8):
        sl = pl.ds(j * 16, 16)
        rowsb[e, sl] = rowsb[e, sl] * nv
      return 0
    lax.fori_loop(0, _CH, scale_body, 0)
    cps2 = [pltpu.async_copy(rowsb.at[pl.ds(128 * r, 128)],
                             acc.at[db.at[r]], sem2, add=True)
            for r in range(2)]
    for cp in cps2:
      cp.wait()
    return 0
  lax.fori_loop(0, _NCHUNK, chunk_body, 0)
  plsc.subcore_barrier()

  pltpu.sync_copy(bias_h, bb)
  bvs = [plsc.load_gather(bb, [cid16, lax.iota(_i32, 16) + 16 * j])
         for j in range(8)]
  for k in range(5):
    base = nbase + 128 * k
    pltpu.sync_copy(acc.at[pl.ds(base, 128)], rowsb.at[pl.ds(0, 128)])
    def ebody(i, _):
      for j in range(8):
        sl = pl.ds(j * 16, 16)
        v = rowsb[i, sl] + bvs[j]
        if relu:
          v = jnp.maximum(v, 0.0)
        rowsb[i, sl] = v
      return 0
    lax.fori_loop(0, 128, ebody, 0)
    @pl.when(cid == 0)
    def _():
      pltpu.sync_copy(rowsb.at[pl.ds(0, 128)], ol_h.at[pl.ds(base, 128)])
    @pl.when(cid == 1)
    def _():
      pltpu.sync_copy(rowsb.at[pl.ds(0, 128)], or_h.at[pl.ds(base, 128)])


def _make_conv_kernel(relu):
  mesh = plsc.VectorSubcoreMesh(core_axis_name="c", subcore_axis_name="s")
  return pl.kernel(
      functools.partial(_conv_body, relu),
      out_type=(jax.ShapeDtypeStruct((_NP, 128), _f32),
                jax.ShapeDtypeStruct((_NP, 128), _f32)),
      mesh=mesh,
      compiler_params=pltpu.CompilerParams(needs_layout_passes=False),
      scratch_types=[
          pltpu.VMEM_SHARED((_NP, 128), _f32),
          pltpu.VMEM((_CH, 128), _f32),
          pltpu.VMEM((2, 128), _i32),
          pltpu.VMEM((2, 128), _i32),
          pltpu.VMEM((2, 128), _i32),
          pltpu.VMEM((_CH,), _f32),
          pltpu.VMEM((2, 128), _f32),
          pltpu.SemaphoreType.DMA,
          pltpu.SemaphoreType.DMA,
      ],
  )


def _mm1_body(x_ref, w_ref, o_ref):
  o_ref[...] = jnp.dot(x_ref[...], w_ref[...],
                       preferred_element_type=_f32)


def _mm2_body(x1_ref, w1_ref, x2_ref, w2_ref, o_ref):
  o_ref[...] = (jnp.dot(x1_ref[...], w1_ref[...],
                        preferred_element_type=_f32) +
                jnp.dot(x2_ref[...], w2_ref[...],
                        preferred_element_type=_f32))


def _mm(x1, w1, x2=None, w2=None):
  bm = 256
  grid = (_NP // bm,)
  kin = w1.shape[0]
  kout = w1.shape[1]
  xspec = pl.BlockSpec((bm, kin), lambda i: (i, 0))
  wspec = pl.BlockSpec((kin, kout), lambda i: (0, 0))
  ospec = pl.BlockSpec((bm, kout), lambda i: (i, 0))
  if x2 is None:
    return pl.pallas_call(
        _mm1_body,
        grid=grid,
        in_specs=[xspec, wspec],
        out_specs=ospec,
        out_shape=jax.ShapeDtypeStruct((_NP, kout), _f32),
    )(x1, w1)
  return pl.pallas_call(
      _mm2_body,
      grid=grid,
      in_specs=[xspec, wspec, xspec, wspec],
      out_specs=ospec,
      out_shape=jax.ShapeDtypeStruct((_NP, kout), _f32),
  )(x1, w1, x2, w2)


_norm_kernel = _make_norm_kernel()
_conv_relu = _make_conv_kernel(True)
_conv_plain = _make_conv_kernel(False)


@jax.jit
def kernel(x, edge_index, edge_attr, W1, b1, W2, b2, W_mu, b_mu, W_std,
           b_std):
  src = edge_index[0]
  dst = edge_index[1]
  pad = _E2P - _E - _N
  loop = jnp.arange(_N, dtype=_i32)
  zpad_i = jnp.zeros((pad,), _i32)
  src2 = jnp.concatenate([src, loop, zpad_i])
  dst2 = jnp.concatenate([dst, loop, zpad_i])
  ew2 = jnp.concatenate([edge_attr, jnp.ones((_N,), _f32),
                         jnp.zeros((pad,), _f32)])
  nrows = _E2P // 128
  src2d = src2.reshape(nrows, 128)
  dst2d = dst2.reshape(nrows, 128)
  ew2d = ew2.reshape(nrows, 128)

  norm2d = _norm_kernel(src2d, dst2d, ew2d)
  norm_f = norm2d.reshape(_E2P)

  x_p = jnp.pad(x, ((0, _NP - _N), (0, 0)))
  b1_2 = b1.reshape(2, 128)
  b2_2 = b2.reshape(2, 128)
  bms = jnp.stack([b_mu, b_std])

  h = _mm(x_p, W1)
  h1l, h1r = _conv_relu(h.reshape(2 * _NP, 128), src2d, dst2d, norm_f,b1_2)

  g = _mm(h1l, W2[:128], h1r, W2[128:])
  hl, hr = _conv_relu(g.reshape(2 * _NP, 128), src2d, dst2d, norm_f,b2_2)

  m = _mm(hl, jnp.concatenate([W_mu, W_std], axis=1)[:128],
          hr, jnp.concatenate([W_mu, W_std], axis=1)[128:])
  mu, std = _conv_plain(m.reshape(2 * _NP, 128), src2d, dst2d, norm_f,bms)
  return (mu[:_N], std[:_N])


# speedup vs baseline: 8.4848x; 1.5725x over previous
import functools

import jax
import jax.numpy as jnp
from jax import lax
from jax.experimental import pallas as pl
from jax.experimental.pallas import tpu as pltpu
from jax.experimental.pallas import tpu_sc as plsc

_N = 10000
_NP = 10240
_E = 320000
_D_IN = 128
_H1 = 256
_H2 = 128

_NT = 16
_NC = 2
_T = 20992
_E2P = _NT * _T
_CH = 256
_NCHUNK = _T // _CH
_RPT = _T // 128
_NODES_PER_TILE = _NP // _NT

_f32 = jnp.float32
_i32 = jnp.int32


def _bcast16(s):
  return jnp.broadcast_to(s, (16,))


def _rsqrt16(d):
  x = jnp.maximum(d, 1e-12)
  i = lax.bitcast_convert_type(x, _i32)
  i = jnp.int32(0x5F3759DF) - (i >> 1)
  y = lax.bitcast_convert_type(i, _f32)
  for _ in range(3):
    y = y * (1.5 - 0.5 * x * y * y)
  return jnp.where(d > 0, y, 0.0)


def _norm_body(src_h, dst_h, ew_h, norm_h,
               deg_sp, dinv_sp, sb, db, wb, nb, dinvb, slab, sem):
  cid = lax.axis_index("c")
  wid = lax.axis_index("s")

  zero16 = jnp.zeros((16,), _f32)
  def zbody(i, _):
    slab[pl.ds(i * 16, 16)] = zero16
    return 0
  lax.fori_loop(0, _NODES_PER_TILE // 16, zbody, 0)
  pltpu.sync_copy(slab, deg_sp.at[pl.ds(wid * _NODES_PER_TILE,
                                        _NODES_PER_TILE)])
  plsc.subcore_barrier()

  def deg_body(c, _):
    row = wid * _RPT + 4 * c
    pltpu.sync_copy(dst_h.at[pl.ds(row, 4)], db)
    pltpu.sync_copy(ew_h.at[pl.ds(row, 4)], wb)
    for r in range(4):
      pltpu.sync_copy(wb.at[r], deg_sp.at[db.at[r]], add=True)
    return 0
  lax.fori_loop(0, _RPT // 4, deg_body, 0)
  plsc.subcore_barrier()

  pltpu.sync_copy(deg_sp.at[pl.ds(wid * _NODES_PER_TILE, _NODES_PER_TILE)],
                  slab)
  def rbody(i, _):
    d = slab[pl.ds(i * 16, 16)]
    slab[pl.ds(i * 16, 16)] = _rsqrt16(d)
    return 0
  lax.fori_loop(0, _NODES_PER_TILE // 16, rbody, 0)
  pltpu.sync_copy(slab, dinv_sp.at[pl.ds(wid * _NODES_PER_TILE,
                                         _NODES_PER_TILE)])
  plsc.subcore_barrier()
  pltpu.sync_copy(dinv_sp, dinvb)

  w32 = wid * _NC + cid
  rows_per_w = (_E2P // 128) // (_NT * _NC)
  def norm_chunk(c, _):
    row = w32 * rows_per_w + 2 * c
    pltpu.sync_copy(src_h.at[pl.ds(row, 2)], sb)
    pltpu.sync_copy(dst_h.at[pl.ds(row, 2)], db.at[pl.ds(0, 2)])
    pltpu.sync_copy(ew_h.at[pl.ds(row, 2)], wb.at[pl.ds(0, 2)])
    for r in range(2):
      for k in range(8):
        sl = pl.ds(k * 16, 16)
        s = sb[r, sl]
        d = db[r, sl]
        w = wb[r, sl]
        gs = plsc.load_gather(dinvb, [s])
        gd = plsc.load_gather(dinvb, [d])
        nb[r, sl] = gs * w * gd
    pltpu.sync_copy(nb, norm_h.at[pl.ds(row, 2)])
    return 0
  lax.fori_loop(0, rows_per_w // 2, norm_chunk, 0)


def _make_norm_kernel():
  mesh = plsc.VectorSubcoreMesh(core_axis_name="c", subcore_axis_name="s")
  nrows = _E2P // 128
  return pl.kernel(
      _norm_body,
      out_type=jax.ShapeDtypeStruct((nrows, 128), _f32),
      mesh=mesh,
      compiler_params=pltpu.CompilerParams(needs_layout_passes=False),
      scratch_types=[
          pltpu.VMEM_SHARED((_NP,), _f32),
          pltpu.VMEM_SHARED((_NP,), _f32),
          pltpu.VMEM((2, 128), _i32),
          pltpu.VMEM((4, 128), _i32),
          pltpu.VMEM((4, 128), _f32),
          pltpu.VMEM((2, 128), _f32),
          pltpu.VMEM((_NP,), _f32),
          pltpu.VMEM((_NODES_PER_TILE,), _f32),
          pltpu.SemaphoreType.DMA,
      ],
  )


_BLK = 64
_BPC = 8
_MC = _T // (_BLK * _BPC)


def _conv_body(relu, h2_h, src_h, dst_h, norm_h, bias_h, ol_h, or_h,
               acc, r0, r1, r2, r3,
               sbA, dbA, nfA, ibA, sbB, dbB, nfB, ibB,
               d0, d1, d2, d3, bb,
               g0, g1, g2, g3, s0, s1, s2, s3, msS, msD, msN):
  cid = lax.axis_index("c")
  wid = lax.axis_index("s")
  cid16 = _bcast16(cid)
  rows = [r0, r1, r2, r3]
  dsc = [d0, d1, d2, d3]
  gs = [g0, g1, g2, g3]
  ss = [s0, s1, s2, s3]
  metas = [(sbA, dbA, nfA, ibA), (sbB, dbB, nfB, ibB)]

  def meta_fire(cv, mp):
    sb_, db_, nf_, _ = metas[mp]
    row = wid * _RPT + 4 * cv
    eb = wid * _T + 512 * cv
    pltpu.async_copy(src_h.at[pl.ds(row, 4)], sb_, msS)
    pltpu.async_copy(dst_h.at[pl.ds(row, 4)], db_, msD)
    pltpu.async_copy(norm_h.at[pl.ds(eb, 512)], nf_, msN)

  def meta_wait(mp):
    sb_, db_, nf_, _ = metas[mp]
    pltpu.make_async_copy(src_h.at[pl.ds(0, 4)], sb_, msS).wait()
    pltpu.make_async_copy(dst_h.at[pl.ds(0, 4)], db_, msD).wait()
    pltpu.make_async_copy(norm_h.at[pl.ds(0, 512)], nf_, msN).wait()

  def idx_compute(mp):
    sb_, _, _, ib_ = metas[mp]
    for b in range(_BPC):
      r = b // 2
      off = 64 * (b % 2)
      for k in range(4):
        ib_[b, pl.ds(16 * k, 16)] = sb_[r, pl.ds(off + 16 * k, 16)] * 2 + cid16

  def fire_gather(mp, rowix, p):
    ib_ = metas[mp][3]
    pltpu.async_copy(h2_h.at[ib_.at[rowix]], rows[p], gs[p])

  def wait_gather(p):
    pltpu.make_async_copy(h2_h.at[ibA.at[0]], rows[p], gs[p]).wait()

  def scale_block(mp, b, p):
    nf_ = metas[mp][2]
    rp = rows[p]
    base = b * _BLK
    def sbody(e, _):
      nv = plsc.load_gather(nf_, [_bcast16(base + e)])
      for j in range(8):
        sl = pl.ds(j * 16, 16)
        rp[e, sl] = rp[e, sl] * nv
      return 0
    lax.fori_loop(0, _BLK, sbody, 0)

  def fire_scatter(mp, b, p):
    db_ = metas[mp][1]
    r = b // 2
    off = 64 * (b % 2)
    for k in range(4):
      dsc[p][0, pl.ds(16 * k, 16)] = db_[r, pl.ds(off + 16 * k, 16)]
    pltpu.async_copy(rows[p], acc.at[dsc[p].at[0]], ss[p], add=True)

  def wait_scatter(p):
    pltpu.make_async_copy(rows[p], acc.at[dsc[p].at[0]], ss[p]).wait()

  def do_chunk(cv, mp, first=False, has_next=True, fire_next2=True):
    mq = 1 - mp
    for b in range(_BPC):
      p = b % 4
      if b == 5 and has_next:
        meta_wait(mq)
        idx_compute(mq)
      wait_gather(p)
      scale_block(mp, b, p)
      fire_scatter(mp, b, p)
      tp = (b + 3) % 4
      last_tail = (not has_next) and b >= 5
      if not last_tail:
        if not (first and b == 0):
          wait_scatter(tp)
        if b < 5:
          fire_gather(mp, b + 3, tp)
        else:
          fire_gather(mq, b - 5, tp)
    if fire_next2:
      meta_fire(cv + 2, mp)

  zero16 = jnp.zeros((16,), _f32)
  def zbody(i, _):
    for j in range(8):
      r0[i, pl.ds(j * 16, 16)] = zero16
    return 0
  lax.fori_loop(0, _BLK, zbody, 0)
  nbase = wid * _NODES_PER_TILE
  for k in range(10):
    pltpu.sync_copy(r0.at[pl.ds(0, _BLK)],
                    acc.at[pl.ds(nbase + _BLK * k, _BLK)])
  plsc.subcore_barrier()

  row0 = wid * _RPT
  eb0 = wid * _T
  pltpu.sync_copy(src_h.at[pl.ds(row0, 4)], sbA)
  pltpu.sync_copy(dst_h.at[pl.ds(row0, 4)], dbA)
  pltpu.sync_copy(norm_h.at[pl.ds(eb0, 512)], nfA)
  idx_compute(0)
  fire_gather(0, 0, 0)
  fire_gather(0, 1, 1)
  fire_gather(0, 2, 2)
  meta_fire(1, 1)

  do_chunk(0, 0, first=True)
  def pair(g, _):
    c1 = 2 * g + 1
    do_chunk(c1, 1)
    do_chunk(c1 + 1, 0)
    return 0
  lax.fori_loop(0, 19, pair, 0)
  do_chunk(39, 1, fire_next2=False)
  do_chunk(40, 0, has_next=False, fire_next2=False)
  for p in range(4):
    wait_scatter(p)
  plsc.subcore_barrier()

  pltpu.sync_copy(bias_h, bb)
  bvs = [plsc.load_gather(bb, [cid16, lax.iota(_i32, 16) + 16 * j])
         for j in range(8)]
  for k in range(10):
    base = nbase + _BLK * k
    pltpu.sync_copy(acc.at[pl.ds(base, _BLK)], r0.at[pl.ds(0, _BLK)])
    def ebody(i, _):
      for j in range(8):
        sl = pl.ds(j * 16, 16)
        v = r0[i, sl] + bvs[j]
        if relu:
          v = jnp.maximum(v, 0.0)
        r0[i, sl] = v
      return 0
    lax.fori_loop(0, _BLK, ebody, 0)
    @pl.when(cid == 0)
    def _():
      pltpu.sync_copy(r0.at[pl.ds(0, _BLK)], ol_h.at[pl.ds(base, _BLK)])
    @pl.when(cid == 1)
    def _():
      pltpu.sync_copy(r0.at[pl.ds(0, _BLK)], or_h.at[pl.ds(base, _BLK)])


def _make_conv_kernel(relu):
  mesh = plsc.VectorSubcoreMesh(core_axis_name="c", subcore_axis_name="s")
  rowbuf = pltpu.VMEM((_BLK, 128), _f32)
  dscbuf = pltpu.VMEM((1, _BLK), _i32)
  sem = pltpu.SemaphoreType.DMA
  return pl.kernel(
      functools.partial(_conv_body, relu),
      out_type=(jax.ShapeDtypeStruct((_NP, 128), _f32),
                jax.ShapeDtypeStruct((_NP, 128), _f32)),
      mesh=mesh,
      compiler_params=pltpu.CompilerParams(needs_layout_passes=False),
      scratch_types=[
          pltpu.VMEM_SHARED((_NP, 128), _f32),
          rowbuf, rowbuf, rowbuf, rowbuf,
          pltpu.VMEM((4, 128), _i32),
          pltpu.VMEM((4, 128), _i32),
          pltpu.VMEM((512,), _f32),
          pltpu.VMEM((_BPC, _BLK), _i32),
          pltpu.VMEM((4, 128), _i32),
          pltpu.VMEM((4, 128), _i32),
          pltpu.VMEM((512,), _f32),
          pltpu.VMEM((_BPC, _BLK), _i32),
          dscbuf, dscbuf, dscbuf, dscbuf,
          pltpu.VMEM((2, 128), _f32),
          sem, sem, sem, sem,
          sem, sem, sem, sem,
          sem, sem, sem,
      ],
  )


def _mm1_body(x_ref, w_ref, o_ref):
  o_ref[...] = jnp.dot(x_ref[...], w_ref[...],
                       preferred_element_type=_f32)


def _mm2_body(x1_ref, w1_ref, x2_ref, w2_ref, o_ref):
  o_ref[...] = (jnp.dot(x1_ref[...], w1_ref[...],
                        preferred_element_type=_f32) +
                jnp.dot(x2_ref[...], w2_ref[...],
                        preferred_element_type=_f32))


def _mm(x1, w1, x2=None, w2=None):
  bm = 256
  grid = (_NP // bm,)
  kin = w1.shape[0]
  kout = w1.shape[1]
  xspec = pl.BlockSpec((bm, kin), lambda i: (i, 0))
  wspec = pl.BlockSpec((kin, kout), lambda i: (0, 0))
  ospec = pl.BlockSpec((bm, kout), lambda i: (i, 0))
  if x2 is None:
    return pl.pallas_call(
        _mm1_body,
        grid=grid,
        in_specs=[xspec, wspec],
        out_specs=ospec,
        out_shape=jax.ShapeDtypeStruct((_NP, kout), _f32),
    )(x1, w1)
  return pl.pallas_call(
      _mm2_body,
      grid=grid,
      in_specs=[xspec, wspec, xspec, wspec],
      out_specs=ospec,
      out_shape=jax.ShapeDtypeStruct((_NP, kout), _f32),
  )(x1, w1, x2, w2)


_norm_kernel = _make_norm_kernel()
_conv_relu = _make_conv_kernel(True)
_conv_plain = _make_conv_kernel(False)


@jax.jit
def kernel(x, edge_index, edge_attr, W1, b1, W2, b2, W_mu, b_mu, W_std,
           b_std):
  src = edge_index[0]
  dst = edge_index[1]
  pad = _E2P - _E - _N
  loop = jnp.arange(_N, dtype=_i32)
  zpad_i = jnp.zeros((pad,), _i32)
  src2 = jnp.concatenate([src, loop, zpad_i])
  dst2 = jnp.concatenate([dst, loop, zpad_i])
  ew2 = jnp.concatenate([edge_attr, jnp.ones((_N,), _f32),
                         jnp.zeros((pad,), _f32)])
  nrows = _E2P // 128
  src2d = src2.reshape(nrows, 128)
  dst2d = dst2.reshape(nrows, 128)
  ew2d = ew2.reshape(nrows, 128)

  norm2d = _norm_kernel(src2d, dst2d, ew2d)
  norm_f = norm2d.reshape(_E2P)

  x_p = jnp.pad(x, ((0, _NP - _N), (0, 0)))
  b1_2 = b1.reshape(2, 128)
  b2_2 = b2.reshape(2, 128)
  bms = jnp.stack([b_mu, b_std])

  h = _mm(x_p, W1)
  h1l, h1r = _conv_relu(h.reshape(2 * _NP, 128), src2d, dst2d, norm_f,b1_2)

  g = _mm(h1l, W2[:128], h1r, W2[128:])
  hl, hr = _conv_relu(g.reshape(2 * _NP, 128), src2d, dst2d, norm_f,b2_2)

  m = _mm(hl, jnp.concatenate([W_mu, W_std], axis=1)[:128],
          hr, jnp.concatenate([W_mu, W_std], axis=1)[128:])
  mu, std = _conv_plain(m.reshape(2 * _NP, 128), src2d, dst2d, norm_f,bms)
  return (mu[:_N], std[:_N])


# speedup vs baseline: 8.5434x; 1.0069x over previous
import functools

import jax
import jax.numpy as jnp
from jax import lax
from jax.experimental import pallas as pl
from jax.experimental.pallas import tpu as pltpu
from jax.experimental.pallas import tpu_sc as plsc

_N = 10000
_NP = 10240
_E = 320000
_D_IN = 128
_H1 = 256
_H2 = 128

_NT = 16
_NC = 2
_T = 20992
_E2P = _NT * _T
_CH = 256
_NCHUNK = _T // _CH
_RPT = _T // 128
_NODES_PER_TILE = _NP // _NT

_f32 = jnp.float32
_i32 = jnp.int32


def _bcast16(s):
  return jnp.broadcast_to(s, (16,))


def _rsqrt16(d):
  x = jnp.maximum(d, 1e-12)
  i = lax.bitcast_convert_type(x, _i32)
  i = jnp.int32(0x5F3759DF) - (i >> 1)
  y = lax.bitcast_convert_type(i, _f32)
  for _ in range(3):
    y = y * (1.5 - 0.5 * x * y * y)
  return jnp.where(d > 0, y, 0.0)


def _norm_body(src_h, dst_h, ew_h, norm_h,
               deg_sp, dinv_sp, sb, db, wb, nb, dinvb, slab, sem):
  cid = lax.axis_index("c")
  wid = lax.axis_index("s")

  zero16 = jnp.zeros((16,), _f32)
  def zbody(i, _):
    slab[pl.ds(i * 16, 16)] = zero16
    return 0
  lax.fori_loop(0, _NODES_PER_TILE // 16, zbody, 0)
  pltpu.sync_copy(slab, deg_sp.at[pl.ds(wid * _NODES_PER_TILE,
                                        _NODES_PER_TILE)])
  plsc.subcore_barrier()

  def deg_body(c, _):
    row = wid * _RPT + 4 * c
    pltpu.sync_copy(dst_h.at[pl.ds(row, 4)], db)
    pltpu.sync_copy(ew_h.at[pl.ds(row, 4)], wb)
    for r in range(4):
      pltpu.sync_copy(wb.at[r], deg_sp.at[db.at[r]], add=True)
    return 0
  lax.fori_loop(0, _RPT // 4, deg_body, 0)
  plsc.subcore_barrier()

  pltpu.sync_copy(deg_sp.at[pl.ds(wid * _NODES_PER_TILE, _NODES_PER_TILE)],
                  slab)
  def rbody(i, _):
    d = slab[pl.ds(i * 16, 16)]
    slab[pl.ds(i * 16, 16)] = _rsqrt16(d)
    return 0
  lax.fori_loop(0, _NODES_PER_TILE // 16, rbody, 0)
  pltpu.sync_copy(slab, dinv_sp.at[pl.ds(wid * _NODES_PER_TILE,
                                         _NODES_PER_TILE)])
  plsc.subcore_barrier()
  pltpu.sync_copy(dinv_sp, dinvb)

  w32 = wid * _NC + cid
  rows_per_w = (_E2P // 128) // (_NT * _NC)
  def norm_chunk(c, _):
    row = w32 * rows_per_w + 2 * c
    pltpu.sync_copy(src_h.at[pl.ds(row, 2)], sb)
    pltpu.sync_copy(dst_h.at[pl.ds(row, 2)], db.at[pl.ds(0, 2)])
    pltpu.sync_copy(ew_h.at[pl.ds(row, 2)], wb.at[pl.ds(0, 2)])
    for r in range(2):
      for k in range(8):
        sl = pl.ds(k * 16, 16)
        s = sb[r, sl]
        d = db[r, sl]
        w = wb[r, sl]
        gs = plsc.load_gather(dinvb, [s])
        gd = plsc.load_gather(dinvb, [d])
        nb[r, sl] = gs * w * gd
    pltpu.sync_copy(nb, norm_h.at[pl.ds(row, 2)])
    return 0
  lax.fori_loop(0, rows_per_w // 2, norm_chunk, 0)


def _make_norm_kernel():
  mesh = plsc.VectorSubcoreMesh(core_axis_name="c", subcore_axis_name="s")
  nrows = _E2P // 128
  return pl.kernel(
      _norm_body,
      out_type=jax.ShapeDtypeStruct((nrows, 128), _f32),
      mesh=mesh,
      compiler_params=pltpu.CompilerParams(needs_layout_passes=False),
      scratch_types=[
          pltpu.VMEM_SHARED((_NP,), _f32),
          pltpu.VMEM_SHARED((_NP,), _f32),
          pltpu.VMEM((2, 128), _i32),
          pltpu.VMEM((4, 128), _i32),
          pltpu.VMEM((4, 128), _f32),
          pltpu.VMEM((2, 128), _f32),
          pltpu.VMEM((_NP,), _f32),
          pltpu.VMEM((_NODES_PER_TILE,), _f32),
          pltpu.SemaphoreType.DMA,
      ],
  )


_BLK = 64
_BPC = 8
_MC = _T // (_BLK * _BPC)


def _conv_body(relu, h2_h, src_h, dst_h, norm_h, bias_h, ol_h, or_h,
               acc, r0, r1, r2, r3,
               sbA, dbA, nfA, ibA, sbB, dbB, nfB, ibB,
               d0, d1, d2, d3, bb,
               g0, g1, g2, g3, s0, s1, s2, s3, msS, msD, msN):
  cid = lax.axis_index("c")
  wid = lax.axis_index("s")
  cid16 = _bcast16(cid)
  rows = [r0, r1, r2, r3]
  dsc = [d0, d1, d2, d3]
  gs = [g0, g1, g2, g3]
  ss = [s0, s1, s2, s3]
  metas = [(sbA, dbA, nfA, ibA), (sbB, dbB, nfB, ibB)]

  def meta_fire(cv, mp):
    sb_, db_, nf_, _ = metas[mp]
    row = wid * _RPT + 4 * cv
    eb = wid * _T + 512 * cv
    pltpu.async_copy(src_h.at[pl.ds(row, 4)], sb_, msS)
    pltpu.async_copy(dst_h.at[pl.ds(row, 4)], db_, msD)
    pltpu.async_copy(norm_h.at[pl.ds(eb, 512)], nf_, msN)

  def meta_wait(mp):
    sb_, db_, nf_, _ = metas[mp]
    pltpu.make_async_copy(src_h.at[pl.ds(0, 4)], sb_, msS).wait()
    pltpu.make_async_copy(dst_h.at[pl.ds(0, 4)], db_, msD).wait()
    pltpu.make_async_copy(norm_h.at[pl.ds(0, 512)], nf_, msN).wait()

  def idx_compute(mp):
    sb_, _, _, ib_ = metas[mp]
    for b in range(_BPC):
      r = b // 2
      off = 64 * (b % 2)
      for k in range(4):
        ib_[b, pl.ds(16 * k, 16)] = sb_[r, pl.ds(off + 16 * k, 16)] * 2 + cid16

  def fire_gather(mp, rowix, p):
    ib_ = metas[mp][3]
    pltpu.async_copy(h2_h.at[ib_.at[rowix]], rows[p], gs[p])

  def wait_gather(p):
    pltpu.make_async_copy(h2_h.at[ibA.at[0]], rows[p], gs[p]).wait()

  def scale_block(mp, b, p):
    nf_ = metas[mp][2]
    rp = rows[p]
    base = b * _BLK
    @plsc.parallel_loop(0, _BLK, unroll=4)
    def sbody(e):
      nv = plsc.load_gather(nf_, [_bcast16(base + e)])
      for j in range(8):
        sl = pl.ds(j * 16, 16)
        rp[e, sl] = rp[e, sl] * nv

  def fire_scatter(mp, b, p):
    db_ = metas[mp][1]
    r = b // 2
    off = 64 * (b % 2)
    for k in range(4):
      dsc[p][0, pl.ds(16 * k, 16)] = db_[r, pl.ds(off + 16 * k, 16)]
    pltpu.async_copy(rows[p], acc.at[dsc[p].at[0]], ss[p], add=True)

  def wait_scatter(p):
    pltpu.make_async_copy(rows[p], acc.at[dsc[p].at[0]], ss[p]).wait()

  def do_chunk(cv, mp, first=False, has_next=True, fire_next2=True):
    mq = 1 - mp
    for b in range(_BPC):
      p = b % 4
      if b == 5 and has_next:
        meta_wait(mq)
        idx_compute(mq)
      wait_gather(p)
      scale_block(mp, b, p)
      fire_scatter(mp, b, p)
      tp = (b + 3) % 4
      last_tail = (not has_next) and b >= 5
      if not last_tail:
        if not (first and b == 0):
          wait_scatter(tp)
        if b < 5:
          fire_gather(mp, b + 3, tp)
        else:
          fire_gather(mq, b - 5, tp)
    if fire_next2:
      meta_fire(cv + 2, mp)

  zero16 = jnp.zeros((16,), _f32)
  def zbody(i, _):
    for j in range(8):
      r0[i, pl.ds(j * 16, 16)] = zero16
    return 0
  lax.fori_loop(0, _BLK, zbody, 0)
  nbase = wid * _NODES_PER_TILE
  for k in range(10):
    pltpu.sync_copy(r0.at[pl.ds(0, _BLK)],
                    acc.at[pl.ds(nbase + _BLK * k, _BLK)])
  plsc.subcore_barrier()

  row0 = wid * _RPT
  eb0 = wid * _T
  pltpu.sync_copy(src_h.at[pl.ds(row0, 4)], sbA)
  pltpu.sync_copy(dst_h.at[pl.ds(row0, 4)], dbA)
  pltpu.sync_copy(norm_h.at[pl.ds(eb0, 512)], nfA)
  idx_compute(0)
  fire_gather(0, 0, 0)
  fire_gather(0, 1, 1)
  fire_gather(0, 2, 2)
  meta_fire(1, 1)

  do_chunk(0, 0, first=True)
  def pair(g, _):
    c1 = 2 * g + 1
    do_chunk(c1, 1)
    do_chunk(c1 + 1, 0)
    return 0
  lax.fori_loop(0, 19, pair, 0)
  do_chunk(39, 1, fire_next2=False)
  do_chunk(40, 0, has_next=False, fire_next2=False)
  for p in range(4):
    wait_scatter(p)
  plsc.subcore_barrier()

  pltpu.sync_copy(bias_h, bb)
  bvs = [plsc.load_gather(bb, [cid16, lax.iota(_i32, 16) + 16 * j])
         for j in range(8)]
  for k in range(10):
    base = nbase + _BLK * k
    pltpu.sync_copy(acc.at[pl.ds(base, _BLK)], r0.at[pl.ds(0, _BLK)])
    def ebody(i, _):
      for j in range(8):
        sl = pl.ds(j * 16, 16)
        v = r0[i, sl] + bvs[j]
        if relu:
          v = jnp.maximum(v, 0.0)
        r0[i, sl] = v
      return 0
    lax.fori_loop(0, _BLK, ebody, 0)
    @pl.when(cid == 0)
    def _():
      pltpu.sync_copy(r0.at[pl.ds(0, _BLK)], ol_h.at[pl.ds(base, _BLK)])
    @pl.when(cid == 1)
    def _():
      pltpu.sync_copy(r0.at[pl.ds(0, _BLK)], or_h.at[pl.ds(base, _BLK)])


def _make_conv_kernel(relu):
  mesh = plsc.VectorSubcoreMesh(core_axis_name="c", subcore_axis_name="s")
  rowbuf = pltpu.VMEM((_BLK, 128), _f32)
  dscbuf = pltpu.VMEM((1, _BLK), _i32)
  sem = pltpu.SemaphoreType.DMA
  return pl.kernel(
      functools.partial(_conv_body, relu),
      out_type=(jax.ShapeDtypeStruct((_NP, 128), _f32),
                jax.ShapeDtypeStruct((_NP, 128), _f32)),
      mesh=mesh,
      compiler_params=pltpu.CompilerParams(needs_layout_passes=False),
      scratch_types=[
          pltpu.VMEM_SHARED((_NP, 128), _f32),
          rowbuf, rowbuf, rowbuf, rowbuf,
          pltpu.VMEM((4, 128), _i32),
          pltpu.VMEM((4, 128), _i32),
          pltpu.VMEM((512,), _f32),
          pltpu.VMEM((_BPC, _BLK), _i32),
          pltpu.VMEM((4, 128), _i32),
          pltpu.VMEM((4, 128), _i32),
          pltpu.VMEM((512,), _f32),
          pltpu.VMEM((_BPC, _BLK), _i32),
          dscbuf, dscbuf, dscbuf, dscbuf,
          pltpu.VMEM((2, 128), _f32),
          sem, sem, sem, sem,
          sem, sem, sem, sem,
          sem, sem, sem,
      ],
  )


def _mm1_body(x_ref, w_ref, o_ref):
  o_ref[...] = jnp.dot(x_ref[...], w_ref[...],
                       preferred_element_type=_f32)


def _mm2_body(x1_ref, w1_ref, x2_ref, w2_ref, o_ref):
  o_ref[...] = (jnp.dot(x1_ref[...], w1_ref[...],
                        preferred_element_type=_f32) +
                jnp.dot(x2_ref[...], w2_ref[...],
                        preferred_element_type=_f32))


def _mm(x1, w1, x2=None, w2=None):
  bm = 256
  grid = (_NP // bm,)
  kin = w1.shape[0]
  kout = w1.shape[1]
  xspec = pl.BlockSpec((bm, kin), lambda i: (i, 0))
  wspec = pl.BlockSpec((kin, kout), lambda i: (0, 0))
  ospec = pl.BlockSpec((bm, kout), lambda i: (i, 0))
  if x2 is None:
    return pl.pallas_call(
        _mm1_body,
        grid=grid,
        in_specs=[xspec, wspec],
        out_specs=ospec,
        out_shape=jax.ShapeDtypeStruct((_NP, kout), _f32),
    )(x1, w1)
  return pl.pallas_call(
      _mm2_body,
      grid=grid,
      in_specs=[xspec, wspec, xspec, wspec],
      out_specs=ospec,
      out_shape=jax.ShapeDtypeStruct((_NP, kout), _f32),
  )(x1, w1, x2, w2)


_norm_kernel = _make_norm_kernel()
_conv_relu = _make_conv_kernel(True)
_conv_plain = _make_conv_kernel(False)


@jax.jit
def kernel(x, edge_index, edge_attr, W1, b1, W2, b2, W_mu, b_mu, W_std,
           b_std):
  src = edge_index[0]
  dst = edge_index[1]
  pad = _E2P - _E - _N
  loop = jnp.arange(_N, dtype=_i32)
  zpad_i = jnp.zeros((pad,), _i32)
  src2 = jnp.concatenate([src, loop, zpad_i])
  dst2 = jnp.concatenate([dst, loop, zpad_i])
  ew2 = jnp.concatenate([edge_attr, jnp.ones((_N,), _f32),
                         jnp.zeros((pad,), _f32)])
  nrows = _E2P // 128
  src2d = src2.reshape(nrows, 128)
  dst2d = dst2.reshape(nrows, 128)
  ew2d = ew2.reshape(nrows, 128)

  norm2d = _norm_kernel(src2d, dst2d, ew2d)
  norm_f = norm2d.reshape(_E2P)

  x_p = jnp.pad(x, ((0, _NP - _N), (0, 0)))
  b1_2 = b1.reshape(2, 128)
  b2_2 = b2.reshape(2, 128)
  bms = jnp.stack([b_mu, b_std])

  h = _mm(x_p, W1)
  h1l, h1r = _conv_relu(h.reshape(2 * _NP, 128), src2d, dst2d, norm_f,b1_2)

  g = _mm(h1l, W2[:128], h1r, W2[128:])
  hl, hr = _conv_relu(g.reshape(2 * _NP, 128), src2d, dst2d, norm_f,b2_2)

  m = _mm(hl, jnp.concatenate([W_mu, W_std], axis=1)[:128],
          hr, jnp.concatenate([W_mu, W_std], axis=1)[128:])
  mu, std = _conv_plain(m.reshape(2 * _NP, 128), src2d, dst2d, norm_f,bms)
  return (mu[:_N], std[:_N])


# speedup vs baseline: 8.9855x; 1.0518x over previous
import functools

import jax
import jax.numpy as jnp
from jax import lax
from jax.experimental import pallas as pl
from jax.experimental.pallas import tpu as pltpu
from jax.experimental.pallas import tpu_sc as plsc

_N = 10000
_NP = 10240
_E = 320000
_D_IN = 128
_H1 = 256
_H2 = 128

_NT = 16
_NC = 2
_T = 20992
_E2P = _NT * _T
_CH = 256
_NCHUNK = _T // _CH
_RPT = _T // 128
_NODES_PER_TILE = _NP // _NT

_f32 = jnp.float32
_i32 = jnp.int32


def _bcast16(s):
  return jnp.broadcast_to(s, (16,))


def _rsqrt16(d):
  x = jnp.maximum(d, 1e-12)
  i = lax.bitcast_convert_type(x, _i32)
  i = jnp.int32(0x5F3759DF) - (i >> 1)
  y = lax.bitcast_convert_type(i, _f32)
  for _ in range(3):
    y = y * (1.5 - 0.5 * x * y * y)
  return jnp.where(d > 0, y, 0.0)


def _norm_body(src_h, dst_h, ew_h, norm_h,
               deg_sp, dinv_sp,
               sbA, sbB, dbA, dbB, dbC, wbA, wbB, wbC, nbA, nbB,
               dinvb, slab,
               mlA, mlB, mlC, scA, scB, scC, stA, stB):
  cid = lax.axis_index("c")
  wid = lax.axis_index("s")
  dbs3 = [dbA, dbB, dbC]
  wbs3 = [wbA, wbB, wbC]
  dbs = [dbA, dbB]
  wbs = [wbA, wbB]
  sbs = [sbA, sbB]
  nbs = [nbA, nbB]
  mls3 = [mlA, mlB, mlC]
  scs3 = [scA, scB, scC]
  mls = [mlA, mlB]
  sts = [stA, stB]

  zero16 = jnp.zeros((16,), _f32)
  def zbody(i, _):
    slab[pl.ds(i * 16, 16)] = zero16
    return 0
  lax.fori_loop(0, _NODES_PER_TILE // 16, zbody, 0)
  pltpu.sync_copy(slab, deg_sp.at[pl.ds(wid * _NODES_PER_TILE,
                                        _NODES_PER_TILE)])
  plsc.subcore_barrier()

  def p1_fire_loads(cv, par):
    row = wid * _RPT + 4 * cv
    pltpu.async_copy(dst_h.at[pl.ds(row, 4)], dbs3[par], mls3[par])
    pltpu.async_copy(ew_h.at[pl.ds(row, 4)], wbs3[par], mls3[par])

  def p1_chunk(cv, par, first=False, fire2=True):
    pltpu.make_async_copy(dst_h.at[pl.ds(0, 4)], dbs3[par],
                          mls3[par]).wait()
    pltpu.make_async_copy(ew_h.at[pl.ds(0, 4)], wbs3[par],
                          mls3[par]).wait()
    for r in range(4):
      pltpu.async_copy(wbs3[par].at[r], deg_sp.at[dbs3[par].at[r]],
                       scs3[par], add=True)
    pv = (par + 2) % 3
    if not first:
      for r in range(4):
        pltpu.make_async_copy(wbs3[pv].at[r], deg_sp.at[dbs3[pv].at[r]],
                              scs3[pv]).wait()
    if fire2:
      p1_fire_loads(cv + 2, pv)

  p1_fire_loads(0, 0)
  p1_fire_loads(1, 1)
  p1_chunk(0, 0, first=True)
  p1_chunk(1, 1)
  def p1_triple(g, _):
    c1 = 3 * g + 2
    p1_chunk(c1, 2)
    p1_chunk(c1 + 1, 0)
    p1_chunk(c1 + 2, 1)
    return 0
  lax.fori_loop(0, 12, p1_triple, 0)
  p1_chunk(38, 2)
  p1_chunk(39, 0, fire2=False)
  p1_chunk(40, 1, fire2=False)
  for r in range(4):
    pltpu.make_async_copy(wbs3[1].at[r], deg_sp.at[dbs3[1].at[r]],
                          scs3[1]).wait()
  plsc.subcore_barrier()

  pltpu.sync_copy(deg_sp.at[pl.ds(wid * _NODES_PER_TILE, _NODES_PER_TILE)],
                  slab)
  def rbody(i, _):
    d = slab[pl.ds(i * 16, 16)]
    slab[pl.ds(i * 16, 16)] = _rsqrt16(d)
    return 0
  lax.fori_loop(0, _NODES_PER_TILE // 16, rbody, 0)
  pltpu.sync_copy(slab, dinv_sp.at[pl.ds(wid * _NODES_PER_TILE,
                                         _NODES_PER_TILE)])
  plsc.subcore_barrier()
  pltpu.sync_copy(dinv_sp, dinvb)

  w32 = wid * _NC + cid
  rows_per_w = (_E2P // 128) // (_NT * _NC)
  nchunk3 = rows_per_w // 2

  def p3_fire_loads(cv, par):
    row = w32 * rows_per_w + 2 * cv
    pltpu.async_copy(src_h.at[pl.ds(row, 2)], sbs[par], mls[par])
    pltpu.async_copy(dst_h.at[pl.ds(row, 2)], dbs[par].at[pl.ds(0, 2)],
                     mls[par])
    pltpu.async_copy(ew_h.at[pl.ds(row, 2)], wbs[par].at[pl.ds(0, 2)],
                     mls[par])

  def p3_wait_loads(par):
    pltpu.make_async_copy(src_h.at[pl.ds(0, 2)], sbs[par], mls[par]).wait()
    pltpu.make_async_copy(dst_h.at[pl.ds(0, 2)], dbs[par].at[pl.ds(0, 2)],
                          mls[par]).wait()
    pltpu.make_async_copy(ew_h.at[pl.ds(0, 2)], wbs[par].at[pl.ds(0, 2)],
                          mls[par]).wait()

  def p3_chunk(cv, par, first=False, fire2=True):
    row = w32 * rows_per_w + 2 * cv
    p3_wait_loads(par)
    if not first:
      pltpu.make_async_copy(nbs[par], norm_h.at[pl.ds(0, 2)],
                            sts[par]).wait()
    for r in range(2):
      for k in range(8):
        sl = pl.ds(k * 16, 16)
        gsv = plsc.load_gather(dinvb, [sbs[par][r, sl]])
        gdv = plsc.load_gather(dinvb, [dbs[par][r, sl]])
        nbs[par][r, sl] = gsv * wbs[par][r, sl] * gdv
    pltpu.async_copy(nbs[par], norm_h.at[pl.ds(row, 2)], sts[par])
    if fire2:
      p3_fire_loads(cv + 2, par)

  p3_fire_loads(0, 0)
  p3_fire_loads(1, 1)
  p3_chunk(0, 0, first=True)
  p3_chunk(1, 1, first=True)
  p3_chunk(2, 0)
  def p3_pair2(g, _):
    c1 = 2 * g + 3
    p3_chunk(c1, 1)
    p3_chunk(c1 + 1, 0)
    return 0
  lax.fori_loop(0, 18, p3_pair2, 0)
  p3_chunk(39, 1, fire2=False)
  p3_chunk(40, 0, fire2=False)
  for par in range(2):
    pltpu.make_async_copy(nbs[par], norm_h.at[pl.ds(0, 2)], sts[par]).wait()


def _make_norm_kernel():
  mesh = plsc.VectorSubcoreMesh(core_axis_name="c", subcore_axis_name="s")
  nrows = _E2P // 128
  ibuf = pltpu.VMEM((4, 128), _i32)
  fbuf = pltpu.VMEM((4, 128), _f32)
  nbuf = pltpu.VMEM((2, 128), _f32)
  sbuf = pltpu.VMEM((2, 128), _i32)
  sem = pltpu.SemaphoreType.DMA
  return pl.kernel(
      _norm_body,
      out_type=jax.ShapeDtypeStruct((nrows, 128), _f32),
      mesh=mesh,
      compiler_params=pltpu.CompilerParams(needs_layout_passes=False),
      scratch_types=[
          pltpu.VMEM_SHARED((_NP,), _f32),
          pltpu.VMEM_SHARED((_NP,), _f32),
          sbuf, sbuf,
          ibuf, ibuf, ibuf,
          fbuf, fbuf, fbuf,
          nbuf, nbuf,
          pltpu.VMEM((_NP,), _f32),
          pltpu.VMEM((_NODES_PER_TILE,), _f32),
          sem, sem, sem,
          sem, sem, sem,
          sem, sem,
      ],
  )


_BLK = 64
_BPC = 8
_MC = _T // (_BLK * _BPC)


def _conv_body(relu, h2_h, src_h, dst_h, norm_h, bias_h, ol_h, or_h,
               acc, r0, r1, r2, r3,
               sbA, dbA, nfA, ibA, sbB, dbB, nfB, ibB,
               d0, d1, d2, d3, bb,
               g0, g1, g2, g3, s0, s1, s2, s3, msS, msD, msN):
  cid = lax.axis_index("c")
  wid = lax.axis_index("s")
  cid16 = _bcast16(cid)
  rows = [r0, r1, r2, r3]
  dsc = [d0, d1, d2, d3]
  gs = [g0, g1, g2, g3]
  ss = [s0, s1, s2, s3]
  metas = [(sbA, dbA, nfA, ibA), (sbB, dbB, nfB, ibB)]

  def meta_fire(cv, mp):
    sb_, db_, nf_, _ = metas[mp]
    row = wid * _RPT + 4 * cv
    eb = wid * _T + 512 * cv
    pltpu.async_copy(src_h.at[pl.ds(row, 4)], sb_, msS)
    pltpu.async_copy(dst_h.at[pl.ds(row, 4)], db_, msD)
    pltpu.async_copy(norm_h.at[pl.ds(eb, 512)], nf_, msN)

  def meta_wait(mp):
    sb_, db_, nf_, _ = metas[mp]
    pltpu.make_async_copy(src_h.at[pl.ds(0, 4)], sb_, msS).wait()
    pltpu.make_async_copy(dst_h.at[pl.ds(0, 4)], db_, msD).wait()
    pltpu.make_async_copy(norm_h.at[pl.ds(0, 512)], nf_, msN).wait()

  def idx_compute(mp):
    sb_, _, _, ib_ = metas[mp]
    for b in range(_BPC):
      r = b // 2
      off = 64 * (b % 2)
      for k in range(4):
        ib_[b, pl.ds(16 * k, 16)] = sb_[r, pl.ds(off + 16 * k, 16)] * 2 + cid16

  def fire_gather(mp, rowix, p):
    ib_ = metas[mp][3]
    pltpu.async_copy(h2_h.at[ib_.at[rowix]], rows[p], gs[p])

  def wait_gather(p):
    pltpu.make_async_copy(h2_h.at[ibA.at[0]], rows[p], gs[p]).wait()

  def scale_block(mp, b, p):
    nf_ = metas[mp][2]
    rp = rows[p]
    base = b * _BLK
    @plsc.parallel_loop(0, _BLK, unroll=4)
    def sbody(e):
      nv = plsc.load_gather(nf_, [_bcast16(base + e)])
      for j in range(8):
        sl = pl.ds(j * 16, 16)
        rp[e, sl] = rp[e, sl] * nv

  def fire_scatter(mp, b, p):
    db_ = metas[mp][1]
    r = b // 2
    off = 64 * (b % 2)
    for k in range(4):
      dsc[p][0, pl.ds(16 * k, 16)] = db_[r, pl.ds(off + 16 * k, 16)]
    pltpu.async_copy(rows[p], acc.at[dsc[p].at[0]], ss[p], add=True)

  def wait_scatter(p):
    pltpu.make_async_copy(rows[p], acc.at[dsc[p].at[0]], ss[p]).wait()

  def do_chunk(cv, mp, first=False, has_next=True, fire_next2=True):
    mq = 1 - mp
    for b in range(_BPC):
      p = b % 4
      if b == 5 and has_next:
        meta_wait(mq)
        idx_compute(mq)
      wait_gather(p)
      scale_block(mp, b, p)
      fire_scatter(mp, b, p)
      tp = (b + 3) % 4
      last_tail = (not has_next) and b >= 5
      if not last_tail:
        if not (first and b == 0):
          wait_scatter(tp)
        if b < 5:
          fire_gather(mp, b + 3, tp)
        else:
          fire_gather(mq, b - 5, tp)
    if fire_next2:
      meta_fire(cv + 2, mp)

  zero16 = jnp.zeros((16,), _f32)
  def zbody(i, _):
    for j in range(8):
      r0[i, pl.ds(j * 16, 16)] = zero16
    return 0
  lax.fori_loop(0, _BLK, zbody, 0)
  nbase = wid * _NODES_PER_TILE
  for k in range(10):
    pltpu.sync_copy(r0.at[pl.ds(0, _BLK)],
                    acc.at[pl.ds(nbase + _BLK * k, _BLK)])
  plsc.subcore_barrier()

  row0 = wid * _RPT
  eb0 = wid * _T
  pltpu.sync_copy(src_h.at[pl.ds(row0, 4)], sbA)
  pltpu.sync_copy(dst_h.at[pl.ds(row0, 4)], dbA)
  pltpu.sync_copy(norm_h.at[pl.ds(eb0, 512)], nfA)
  idx_compute(0)
  fire_gather(0, 0, 0)
  fire_gather(0, 1, 1)
  fire_gather(0, 2, 2)
  meta_fire(1, 1)

  do_chunk(0, 0, first=True)
  def pair(g, _):
    c1 = 2 * g + 1
    do_chunk(c1, 1)
    do_chunk(c1 + 1, 0)
    return 0
  lax.fori_loop(0, 19, pair, 0)
  do_chunk(39, 1, fire_next2=False)
  do_chunk(40, 0, has_next=False, fire_next2=False)
  for p in range(4):
    wait_scatter(p)
  plsc.subcore_barrier()

  pltpu.sync_copy(bias_h, bb)
  bvs = [plsc.load_gather(bb, [cid16, lax.iota(_i32, 16) + 16 * j])
         for j in range(8)]
  for k in range(10):
    base = nbase + _BLK * k
    pltpu.sync_copy(acc.at[pl.ds(base, _BLK)], r0.at[pl.ds(0, _BLK)])
    def ebody(i, _):
      for j in range(8):
        sl = pl.ds(j * 16, 16)
        v = r0[i, sl] + bvs[j]
        if relu:
          v = jnp.maximum(v, 0.0)
        r0[i, sl] = v
      return 0
    lax.fori_loop(0, _BLK, ebody, 0)
    @pl.when(cid == 0)
    def _():
      pltpu.sync_copy(r0.at[pl.ds(0, _BLK)], ol_h.at[pl.ds(base, _BLK)])
    @pl.when(cid == 1)
    def _():
      pltpu.sync_copy(r0.at[pl.ds(0, _BLK)], or_h.at[pl.ds(base, _BLK)])


def _make_conv_kernel(relu):
  mesh = plsc.VectorSubcoreMesh(core_axis_name="c", subcore_axis_name="s")
  rowbuf = pltpu.VMEM((_BLK, 128), _f32)
  dscbuf = pltpu.VMEM((1, _BLK), _i32)
  sem = pltpu.SemaphoreType.DMA
  return pl.kernel(
      functools.partial(_conv_body, relu),
      out_type=(jax.ShapeDtypeStruct((_NP, 128), _f32),
                jax.ShapeDtypeStruct((_NP, 128), _f32)),
      mesh=mesh,
      compiler_params=pltpu.CompilerParams(needs_layout_passes=False),
      scratch_types=[
          pltpu.VMEM_SHARED((_NP, 128), _f32),
          rowbuf, rowbuf, rowbuf, rowbuf,
          pltpu.VMEM((4, 128), _i32),
          pltpu.VMEM((4, 128), _i32),
          pltpu.VMEM((512,), _f32),
          pltpu.VMEM((_BPC, _BLK), _i32),
          pltpu.VMEM((4, 128), _i32),
          pltpu.VMEM((4, 128), _i32),
          pltpu.VMEM((512,), _f32),
          pltpu.VMEM((_BPC, _BLK), _i32),
          dscbuf, dscbuf, dscbuf, dscbuf,
          pltpu.VMEM((2, 128), _f32),
          sem, sem, sem, sem,
          sem, sem, sem, sem,
          sem, sem, sem,
      ],
  )


def _mm1_body(x_ref, w_ref, o_ref):
  o_ref[...] = jnp.dot(x_ref[...], w_ref[...],
                       preferred_element_type=_f32)


def _mm2_body(x1_ref, w1_ref, x2_ref, w2_ref, o_ref):
  o_ref[...] = (jnp.dot(x1_ref[...], w1_ref[...],
                        preferred_element_type=_f32) +
                jnp.dot(x2_ref[...], w2_ref[...],
                        preferred_element_type=_f32))


def _mm(x1, w1, x2=None, w2=None):
  bm = 256
  grid = (_NP // bm,)
  kin = w1.shape[0]
  kout = w1.shape[1]
  xspec = pl.BlockSpec((bm, kin), lambda i: (i, 0))
  wspec = pl.BlockSpec((kin, kout), lambda i: (0, 0))
  ospec = pl.BlockSpec((bm, kout), lambda i: (i, 0))
  if x2 is None:
    return pl.pallas_call(
        _mm1_body,
        grid=grid,
        in_specs=[xspec, wspec],
        out_specs=ospec,
        out_shape=jax.ShapeDtypeStruct((_NP, kout), _f32),
    )(x1, w1)
  return pl.pallas_call(
      _mm2_body,
      grid=grid,
      in_specs=[xspec, wspec, xspec, wspec],
      out_specs=ospec,
      out_shape=jax.ShapeDtypeStruct((_NP, kout), _f32),
  )(x1, w1, x2, w2)


_norm_kernel = _make_norm_kernel()
_conv_relu = _make_conv_kernel(True)
_conv_plain = _make_conv_kernel(False)


@jax.jit
def kernel(x, edge_index, edge_attr, W1, b1, W2, b2, W_mu, b_mu, W_std,
           b_std):
  src = edge_index[0]
  dst = edge_index[1]
  pad = _E2P - _E - _N
  loop = jnp.arange(_N, dtype=_i32)
  zpad_i = jnp.zeros((pad,), _i32)
  src2 = jnp.concatenate([src, loop, zpad_i])
  dst2 = jnp.concatenate([dst, loop, zpad_i])
  ew2 = jnp.concatenate([edge_attr, jnp.ones((_N,), _f32),
                         jnp.zeros((pad,), _f32)])
  nrows = _E2P // 128
  src2d = src2.reshape(nrows, 128)
  dst2d = dst2.reshape(nrows, 128)
  ew2d = ew2.reshape(nrows, 128)

  norm2d = _norm_kernel(src2d, dst2d, ew2d)
  norm_f = norm2d.reshape(_E2P)

  x_p = jnp.pad(x, ((0, _NP - _N), (0, 0)))
  b1_2 = b1.reshape(2, 128)
  b2_2 = b2.reshape(2, 128)
  bms = jnp.stack([b_mu, b_std])

  h = _mm(x_p, W1)
  h1l, h1r = _conv_relu(h.reshape(2 * _NP, 128), src2d, dst2d, norm_f,b1_2)

  g = _mm(h1l, W2[:128], h1r, W2[128:])
  hl, hr = _conv_relu(g.reshape(2 * _NP, 128), src2d, dst2d, norm_f,b2_2)

  m = _mm(hl, jnp.concatenate([W_mu, W_std], axis=1)[:128],
          hr, jnp.concatenate([W_mu, W_std], axis=1)[128:])
  mu, std = _conv_plain(m.reshape(2 * _NP, 128), src2d, dst2d, norm_f,bms)
  return (mu[:_N], std[:_N])
